# baseline (speedup 1.0000x reference)
import functools

import jax
import jax.numpy as jnp
from jax import lax
from jax.experimental import pallas as pl
from jax.experimental.pallas import tpu as pltpu
from jax.experimental.pallas import tpu_sc as plsc

_B = 16384
_D = 32
_NC = 2
_NS = 16
_NW = _NC * _NS
_BPW = _B // _NW
_CHUNK = 128
_NCHUNK = _BPW // _CHUNK
_L = 16
_NQ = 13
_NREL = 100

_KG_LAMBDA = 1e-05
_CF_LAMBDA = 1e-05


def _fetch_rows(table_hbm, idx_ref, ce0, buf, sem):
  def issue(g, carry):
    e0 = g * _L
    vec = idx_ref[pl.ds(ce0 + e0, _L)]
    for l in range(_L):
      pltpu.async_copy(table_hbm.at[vec[l]], buf.at[e0 + l], sem)
    return carry

  lax.fori_loop(0, _CHUNK // _L, issue, jnp.int32(0))


def _drain_rows(table_hbm, buf, sem):
  def drain(e, carry):
    pltpu.make_async_copy(table_hbm.at[0], buf.at[e], sem).wait()
    return carry

  lax.fori_loop(0, _CHUNK, drain, jnp.int32(0))


def _sc_stage1(user_hbm, item_hbm, entity_hbm, relation_hbm,
               uid_hbm, ipid_hbm, inid_hbm, h_hbm, r_hbm, pt_hbm, nt_hbm,
               out_hbm,
               idx_u, idx_ip, idx_in, idx_h, idx_r, idx_pt, idx_nt,
               buf_a, buf_b, buf_c, buf_d, buf_e, buf_f, rel_buf,
               stage, sem):
  wid = lax.axis_index("s") * _NC + lax.axis_index("c")
  base = wid * _BPW

  for src, dst in ((h_hbm, idx_h), (r_hbm, idx_r), (pt_hbm, idx_pt),
                   (nt_hbm, idx_nt), (uid_hbm, idx_u),
                   (ipid_hbm, idx_ip), (inid_hbm, idx_in)):
    pltpu.sync_copy(src.at[pl.ds(base, _BPW)], dst)

  def rel_issue(e, carry):
    pltpu.async_copy(relation_hbm.at[e], rel_buf.at[e], sem)
    return carry

  lax.fori_loop(0, _NREL, rel_issue, jnp.int32(0))

  def rel_drain(e, carry):
    pltpu.make_async_copy(relation_hbm.at[0], rel_buf.at[e], sem).wait()
    return carry

  lax.fori_loop(0, _NREL, rel_drain, jnp.int32(0))

  iota = lax.iota(jnp.int32, _L)
  zeros = jnp.zeros((_L,), jnp.float32)

  for c in range(_NCHUNK):
    ce0 = c * _CHUNK

    _fetch_rows(entity_hbm, idx_h, ce0, buf_a, sem)
    _fetch_rows(entity_hbm, idx_pt, ce0, buf_c, sem)
    _fetch_rows(entity_hbm, idx_nt, ce0, buf_d, sem)
    _fetch_rows(user_hbm, idx_u, ce0, buf_b, sem)
    _fetch_rows(item_hbm, idx_ip, ce0, buf_e, sem)
    _fetch_rows(item_hbm, idx_in, ce0, buf_f, sem)
    _drain_rows(entity_hbm, buf_a, sem)
    _drain_rows(entity_hbm, buf_c, sem)
    _drain_rows(entity_hbm, buf_d, sem)

    def kg_body(g, carry):
      e0 = g * _L
      rvec = idx_r[pl.ds(ce0 + e0, _L)]
      acc = [zeros] * 8
      for l in range(_L):
        e = e0 + l
        rid = rvec[l]
        h0 = buf_a[e, pl.ds(0, _L)]
        h1 = buf_a[e, pl.ds(_L, _L)]
        rv0 = rel_buf[rid, pl.ds(0, _L)]
        rv1 = rel_buf[rid, pl.ds(_L, _L)]
        p0 = buf_c[e, pl.ds(0, _L)]
        p1 = buf_c[e, pl.ds(_L, _L)]
        n0 = buf_d[e, pl.ds(0, _L)]
        n1 = buf_d[e, pl.ds(_L, _L)]
        lane = iota == l
        vals = (jnp.sum(h0 * h0 + h1 * h1),
                jnp.sum(rv0 * rv0 + rv1 * rv1),
                jnp.sum(p0 * p0 + p1 * p1),
                jnp.sum(n0 * n0 + n1 * n1),
                jnp.sum(h0 * p0 + h1 * p1),
                jnp.sum(rv0 * p0 + rv1 * p1),
                jnp.sum(h0 * n0 + h1 * n1),
                jnp.sum(rv0 * n0 + rv1 * n1))
        acc = [jnp.where(lane, v, a) for v, a in zip(vals, acc)]
      for q in range(8):
        stage[q, pl.ds(ce0 + e0, _L)] = acc[q]
      return carry

    lax.fori_loop(0, _CHUNK // _L, kg_body, jnp.int32(0))

    _fetch_rows(entity_hbm, idx_ip, ce0, buf_c, sem)
    _fetch_rows(entity_hbm, idx_in, ce0, buf_d, sem)
    _drain_rows(user_hbm, buf_b, sem)
    _drain_rows(item_hbm, buf_e, sem)
    _drain_rows(item_hbm, buf_f, sem)
    _drain_rows(entity_hbm, buf_c, sem)
    _drain_rows(entity_hbm, buf_d, sem)

    def cf_body(g, carry):
      e0 = g * _L
      acc = [zeros] * 5
      for l in range(_L):
        e = e0 + l
        u0 = buf_b[e, pl.ds(0, _L)]
        u1 = buf_b[e, pl.ds(_L, _L)]
        ipcf0 = buf_e[e, pl.ds(0, _L)] * buf_c[e, pl.ds(0, _L)]
        ipcf1 = buf_e[e, pl.ds(_L, _L)] * buf_c[e, pl.ds(_L, _L)]
        incf0 = buf_f[e, pl.ds(0, _L)] * buf_d[e, pl.ds(0, _L)]
        incf1 = buf_f[e, pl.ds(_L, _L)] * buf_d[e, pl.ds(_L, _L)]
        lane = iota == l
        vals = (jnp.sum(u0 * ipcf0 + u1 * ipcf1),
                jnp.sum(u0 * incf0 + u1 * incf1),
                jnp.sum(u0 * u0 + u1 * u1),
                jnp.sum(ipcf0 * ipcf0 + ipcf1 * ipcf1),
                jnp.sum(incf0 * incf0 + incf1 * incf1))
        acc = [jnp.where(lane, v, a) for v, a in zip(vals, acc)]
      for q in range(5):
        stage[8 + q, pl.ds(ce0 + e0, _L)] = acc[q]
      return carry

    lax.fori_loop(0, _CHUNK // _L, cf_body, jnp.int32(0))

  pltpu.sync_copy(stage, out_hbm.at[wid])


def _tc_stage2(x_ref, o_ref):
  def q(i):
    return x_ref[:, i, :]

  h2, r2, p2, n2 = q(0), q(1), q(2), q(3)
  hp, rp, hn, rn = q(4), q(5), q(6), q(7)
  ps, ns, u2, ip2, in2 = q(8), q(9), q(10), q(11), q(12)

  diff = 2.0 * (hp * lax.rsqrt(h2 * p2) + rp * lax.rsqrt(r2 * p2)
                - hn * lax.rsqrt(h2 * n2) - rn * lax.rsqrt(r2 * n2))
  kg_loss = jnp.mean(-jax.nn.log_sigmoid(diff))
  kg_total = kg_loss + _KG_LAMBDA * 2.0

  cf_loss = jnp.mean(-jnp.log(1e-10 + jax.nn.sigmoid(ps - ns)))
  cf_l2 = 0.5 * (jnp.mean(u2) + jnp.mean(ip2) + jnp.mean(in2))
  cf_total = cf_loss + _CF_LAMBDA * cf_l2

  o_ref[0, 0] = kg_total + cf_total


@jax.jit
def kernel(user_embed, item_embed, entity_embed, relation_embed,
           user_ids, item_pos_ids, item_neg_ids, h, r, pos_t, neg_t):
  ids = [jnp.asarray(a, jnp.int32)
         for a in (user_ids, item_pos_ids, item_neg_ids, h, r, pos_t, neg_t)]

  mesh = plsc.VectorSubcoreMesh(core_axis_name="c", subcore_axis_name="s")
  stage1 = pl.kernel(
      _sc_stage1,
      out_type=jax.ShapeDtypeStruct((_NW, _NQ, _BPW), jnp.float32),
      mesh=mesh,
      scratch_types=(
          [pltpu.VMEM((_BPW,), jnp.int32)] * 7
          + [pltpu.VMEM((_CHUNK, _D), jnp.float32)] * 6
          + [pltpu.VMEM((_NREL, _D), jnp.float32),
             pltpu.VMEM((_NQ, _BPW), jnp.float32),
             pltpu.SemaphoreType.DMA]),
      compiler_params=pltpu.CompilerParams(needs_layout_passes=False),
  )
  inter = stage1(user_embed, item_embed, entity_embed, relation_embed, *ids)

  out = pl.pallas_call(
      _tc_stage2,
      out_shape=jax.ShapeDtypeStruct((1, 1), jnp.float32),
      out_specs=pl.BlockSpec(memory_space=pltpu.SMEM),
  )(inter)
  return out[0, 0]


# speedup vs baseline: 1.0110x; 1.0110x over previous
import functools

import jax
import jax.numpy as jnp
from jax import lax
from jax.experimental import pallas as pl
from jax.experimental.pallas import tpu as pltpu
from jax.experimental.pallas import tpu_sc as plsc

_B = 16384
_D = 32
_NC = 2
_NS = 16
_NW = _NC * _NS
_BPW = _B // _NW
_CHUNK = 128
_NCHUNK = _BPW // _CHUNK
_L = 16
_NQ = 13
_NREL = 100

_KG_LAMBDA = 1e-05
_CF_LAMBDA = 1e-05


def _fetch_rows(table_hbm, idx_ref, ce0, buf, sem, sem2):
  def issue(g, carry):
    e0 = g * _L
    vec = idx_ref[pl.ds(ce0 + e0, _L)]
    for l in range(_L):
      pltpu.async_copy(table_hbm.at[vec[l]], buf.at[e0 + l],
                       sem if l % 2 == 0 else sem2)
    return carry

  lax.fori_loop(0, _CHUNK // _L, issue, jnp.int32(0))


def _drain_rows(table_hbm, buf, sem, sem2):
  def drain(e, carry):
    pltpu.make_async_copy(table_hbm.at[0], buf.at[2 * e], sem).wait()
    pltpu.make_async_copy(table_hbm.at[0], buf.at[2 * e + 1], sem2).wait()
    return carry

  lax.fori_loop(0, _CHUNK // 2, drain, jnp.int32(0))


def _sc_stage1(user_hbm, item_hbm, entity_hbm, relation_hbm,
               uid_hbm, ipid_hbm, inid_hbm, h_hbm, r_hbm, pt_hbm, nt_hbm,
               out_hbm,
               idx_u, idx_ip, idx_in, idx_h, idx_r, idx_pt, idx_nt,
               buf_a, buf_b, buf_c, buf_d, buf_e, buf_f, rel_buf,
               stage, sem, sem2):
  wid = lax.axis_index("s") * _NC + lax.axis_index("c")
  base = wid * _BPW

  for src, dst in ((h_hbm, idx_h), (r_hbm, idx_r), (pt_hbm, idx_pt),
                   (nt_hbm, idx_nt), (uid_hbm, idx_u),
                   (ipid_hbm, idx_ip), (inid_hbm, idx_in)):
    pltpu.sync_copy(src.at[pl.ds(base, _BPW)], dst)

  def rel_issue(e, carry):
    pltpu.async_copy(relation_hbm.at[e], rel_buf.at[e], sem)
    return carry

  lax.fori_loop(0, _NREL, rel_issue, jnp.int32(0))

  def rel_drain(e, carry):
    pltpu.make_async_copy(relation_hbm.at[0], rel_buf.at[e], sem).wait()
    return carry

  lax.fori_loop(0, _NREL, rel_drain, jnp.int32(0))

  iota = lax.iota(jnp.int32, _L)
  zeros = jnp.zeros((_L,), jnp.float32)

  for c in range(_NCHUNK):
    ce0 = c * _CHUNK

    _fetch_rows(entity_hbm, idx_h, ce0, buf_a, sem, sem2)
    _fetch_rows(entity_hbm, idx_pt, ce0, buf_c, sem, sem2)
    _fetch_rows(entity_hbm, idx_nt, ce0, buf_d, sem, sem2)
    _fetch_rows(user_hbm, idx_u, ce0, buf_b, sem, sem2)
    _fetch_rows(item_hbm, idx_ip, ce0, buf_e, sem, sem2)
    _fetch_rows(item_hbm, idx_in, ce0, buf_f, sem, sem2)
    _drain_rows(entity_hbm, buf_a, sem, sem2)
    _drain_rows(entity_hbm, buf_c, sem, sem2)
    _drain_rows(entity_hbm, buf_d, sem, sem2)

    def kg_body(g, carry):
      e0 = g * _L
      rvec = idx_r[pl.ds(ce0 + e0, _L)]
      acc = [zeros] * 8
      for l in range(_L):
        e = e0 + l
        rid = rvec[l]
        h0 = buf_a[e, pl.ds(0, _L)]
        h1 = buf_a[e, pl.ds(_L, _L)]
        rv0 = rel_buf[rid, pl.ds(0, _L)]
        rv1 = rel_buf[rid, pl.ds(_L, _L)]
        p0 = buf_c[e, pl.ds(0, _L)]
        p1 = buf_c[e, pl.ds(_L, _L)]
        n0 = buf_d[e, pl.ds(0, _L)]
        n1 = buf_d[e, pl.ds(_L, _L)]
        lane = iota == l
        vals = (jnp.sum(h0 * h0 + h1 * h1),
                jnp.sum(rv0 * rv0 + rv1 * rv1),
                jnp.sum(p0 * p0 + p1 * p1),
                jnp.sum(n0 * n0 + n1 * n1),
                jnp.sum(h0 * p0 + h1 * p1),
                jnp.sum(rv0 * p0 + rv1 * p1),
                jnp.sum(h0 * n0 + h1 * n1),
                jnp.sum(rv0 * n0 + rv1 * n1))
        acc = [jnp.where(lane, v, a) for v, a in zip(vals, acc)]
      for q in range(8):
        stage[q, pl.ds(ce0 + e0, _L)] = acc[q]
      return carry

    lax.fori_loop(0, _CHUNK // _L, kg_body, jnp.int32(0))

    _fetch_rows(entity_hbm, idx_ip, ce0, buf_c, sem, sem2)
    _fetch_rows(entity_hbm, idx_in, ce0, buf_d, sem, sem2)
    _drain_rows(user_hbm, buf_b, sem, sem2)
    _drain_rows(item_hbm, buf_e, sem, sem2)
    _drain_rows(item_hbm, buf_f, sem, sem2)
    _drain_rows(entity_hbm, buf_c, sem, sem2)
    _drain_rows(entity_hbm, buf_d, sem, sem2)

    def cf_body(g, carry):
      e0 = g * _L
      acc = [zeros] * 5
      for l in range(_L):
        e = e0 + l
        u0 = buf_b[e, pl.ds(0, _L)]
        u1 = buf_b[e, pl.ds(_L, _L)]
        ipcf0 = buf_e[e, pl.ds(0, _L)] * buf_c[e, pl.ds(0, _L)]
        ipcf1 = buf_e[e, pl.ds(_L, _L)] * buf_c[e, pl.ds(_L, _L)]
        incf0 = buf_f[e, pl.ds(0, _L)] * buf_d[e, pl.ds(0, _L)]
        incf1 = buf_f[e, pl.ds(_L, _L)] * buf_d[e, pl.ds(_L, _L)]
        lane = iota == l
        vals = (jnp.sum(u0 * ipcf0 + u1 * ipcf1),
                jnp.sum(u0 * incf0 + u1 * incf1),
                jnp.sum(u0 * u0 + u1 * u1),
                jnp.sum(ipcf0 * ipcf0 + ipcf1 * ipcf1),
                jnp.sum(incf0 * incf0 + incf1 * incf1))
        acc = [jnp.where(lane, v, a) for v, a in zip(vals, acc)]
      for q in range(5):
        stage[8 + q, pl.ds(ce0 + e0, _L)] = acc[q]
      return carry

    lax.fori_loop(0, _CHUNK // _L, cf_body, jnp.int32(0))

  pltpu.sync_copy(stage, out_hbm.at[wid])


def _tc_stage2(x_ref, o_ref):
  def q(i):
    return x_ref[:, i, :]

  h2, r2, p2, n2 = q(0), q(1), q(2), q(3)
  hp, rp, hn, rn = q(4), q(5), q(6), q(7)
  ps, ns, u2, ip2, in2 = q(8), q(9), q(10), q(11), q(12)

  diff = 2.0 * (hp * lax.rsqrt(h2 * p2) + rp * lax.rsqrt(r2 * p2)
                - hn * lax.rsqrt(h2 * n2) - rn * lax.rsqrt(r2 * n2))
  kg_loss = jnp.mean(-jax.nn.log_sigmoid(diff))
  kg_total = kg_loss + _KG_LAMBDA * 2.0

  cf_loss = jnp.mean(-jnp.log(1e-10 + jax.nn.sigmoid(ps - ns)))
  cf_l2 = 0.5 * (jnp.mean(u2) + jnp.mean(ip2) + jnp.mean(in2))
  cf_total = cf_loss + _CF_LAMBDA * cf_l2

  o_ref[0, 0] = kg_total + cf_total


@jax.jit
def kernel(user_embed, item_embed, entity_embed, relation_embed,
           user_ids, item_pos_ids, item_neg_ids, h, r, pos_t, neg_t):
  ids = [jnp.asarray(a, jnp.int32)
         for a in (user_ids, item_pos_ids, item_neg_ids, h, r, pos_t, neg_t)]

  mesh = plsc.VectorSubcoreMesh(core_axis_name="c", subcore_axis_name="s")
  stage1 = pl.kernel(
      _sc_stage1,
      out_type=jax.ShapeDtypeStruct((_NW, _NQ, _BPW), jnp.float32),
      mesh=mesh,
      scratch_types=(
          [pltpu.VMEM((_BPW,), jnp.int32)] * 7
          + [pltpu.VMEM((_CHUNK, _D), jnp.float32)] * 6
          + [pltpu.VMEM((_NREL, _D), jnp.float32),
             pltpu.VMEM((_NQ, _BPW), jnp.float32),
             pltpu.SemaphoreType.DMA, pltpu.SemaphoreType.DMA]),
      compiler_params=pltpu.CompilerParams(needs_layout_passes=False),
  )
  inter = stage1(user_embed, item_embed, entity_embed, relation_embed, *ids)

  out = pl.pallas_call(
      _tc_stage2,
      out_shape=jax.ShapeDtypeStruct((1, 1), jnp.float32),
      out_specs=pl.BlockSpec(memory_space=pltpu.SMEM),
  )(inter)
  return out[0, 0]


# speedup vs baseline: 1.0111x; 1.0000x over previous
import jax
import jax.numpy as jnp
from jax import lax
from jax.experimental import pallas as pl
from jax.experimental.pallas import tpu as pltpu
from jax.experimental.pallas import tpu_sc as plsc

_B = 16384
_D = 32
_NC = 2
_NS = 16
_NW = _NC * _NS
_BPW = _B // _NW
_CHUNK = 128
_NCHUNK = _BPW // _CHUNK
_L = 16
_NQ = 13
_NREL = 100

_KG_LAMBDA = 1e-05
_CF_LAMBDA = 1e-05


def _fetch_rows(table_hbm, idx_ref, ce0, buf, sem, sem2):
  def issue(g, carry):
    e0 = g * _L
    vec = idx_ref[pl.ds(ce0 + e0, _L)]
    for l in range(_L):
      pltpu.async_copy(table_hbm.at[vec[l]], buf.at[e0 + l],
                       sem if l % 2 == 0 else sem2)
    return carry

  lax.fori_loop(0, _CHUNK // _L, issue, jnp.int32(0))


def _drain_rows(table_hbm, buf, sem, sem2):
  def drain(e, carry):
    pltpu.make_async_copy(table_hbm.at[0], buf.at[2 * e], sem).wait()
    pltpu.make_async_copy(table_hbm.at[0], buf.at[2 * e + 1], sem2).wait()
    return carry

  lax.fori_loop(0, _CHUNK // 2, drain, jnp.int32(0))


def _sc_stage1(user_hbm, item_hbm, entity_hbm, relation_hbm,
               uid_hbm, ipid_hbm, inid_hbm, h_hbm, r_hbm, pt_hbm, nt_hbm,
               out_hbm,
               idx_u, idx_ip, idx_in, idx_h, idx_r, idx_pt, idx_nt,
               buf_a, buf_b, buf_c, buf_d, buf_e, buf_f, rel_buf,
               stage, sem, sem2):
  wid = lax.axis_index("s") * _NC + lax.axis_index("c")
  base = wid * _BPW

  for src, dst in ((h_hbm, idx_h), (r_hbm, idx_r), (pt_hbm, idx_pt),
                   (nt_hbm, idx_nt), (uid_hbm, idx_u),
                   (ipid_hbm, idx_ip), (inid_hbm, idx_in)):
    pltpu.sync_copy(src.at[pl.ds(base, _BPW)], dst)

  def rel_issue(e, carry):
    pltpu.async_copy(relation_hbm.at[e], rel_buf.at[e], sem)
    return carry

  lax.fori_loop(0, _NREL, rel_issue, jnp.int32(0))

  def rel_drain(e, carry):
    pltpu.make_async_copy(relation_hbm.at[0], rel_buf.at[e], sem).wait()
    return carry

  lax.fori_loop(0, _NREL, rel_drain, jnp.int32(0))

  iota = lax.iota(jnp.int32, _L)
  zeros = jnp.zeros((_L,), jnp.float32)

  for c in range(_NCHUNK):
    ce0 = c * _CHUNK

    _fetch_rows(entity_hbm, idx_h, ce0, buf_a, sem, sem2)
    _fetch_rows(entity_hbm, idx_pt, ce0, buf_c, sem, sem2)
    _fetch_rows(entity_hbm, idx_nt, ce0, buf_d, sem, sem2)
    _fetch_rows(user_hbm, idx_u, ce0, buf_b, sem, sem2)
    _fetch_rows(item_hbm, idx_ip, ce0, buf_e, sem, sem2)
    _fetch_rows(item_hbm, idx_in, ce0, buf_f, sem, sem2)
    _drain_rows(entity_hbm, buf_a, sem, sem2)
    _drain_rows(entity_hbm, buf_c, sem, sem2)
    _drain_rows(entity_hbm, buf_d, sem, sem2)

    def kg_body(g, carry):
      e0 = g * _L
      rvec = idx_r[pl.ds(ce0 + e0, _L)]
      acc = [zeros] * 8
      for l in range(_L):
        e = e0 + l
        rid = rvec[l]
        h0 = buf_a[e, pl.ds(0, _L)]
        h1 = buf_a[e, pl.ds(_L, _L)]
        rv0 = rel_buf[rid, pl.ds(0, _L)]
        rv1 = rel_buf[rid, pl.ds(_L, _L)]
        p0 = buf_c[e, pl.ds(0, _L)]
        p1 = buf_c[e, pl.ds(_L, _L)]
        n0 = buf_d[e, pl.ds(0, _L)]
        n1 = buf_d[e, pl.ds(_L, _L)]
        lane = iota == l
        vals = (jnp.sum(h0 * h0 + h1 * h1),
                jnp.sum(rv0 * rv0 + rv1 * rv1),
                jnp.sum(p0 * p0 + p1 * p1),
                jnp.sum(n0 * n0 + n1 * n1),
                jnp.sum(h0 * p0 + h1 * p1),
                jnp.sum(rv0 * p0 + rv1 * p1),
                jnp.sum(h0 * n0 + h1 * n1),
                jnp.sum(rv0 * n0 + rv1 * n1))
        acc = [jnp.where(lane, v, a) for v, a in zip(vals, acc)]
      for q in range(8):
        stage[q, pl.ds(ce0 + e0, _L)] = acc[q]
      return carry

    lax.fori_loop(0, _CHUNK // _L, kg_body, jnp.int32(0))

    _fetch_rows(entity_hbm, idx_ip, ce0, buf_c, sem, sem2)
    _fetch_rows(entity_hbm, idx_in, ce0, buf_d, sem, sem2)
    _drain_rows(user_hbm, buf_b, sem, sem2)
    _drain_rows(item_hbm, buf_e, sem, sem2)
    _drain_rows(item_hbm, buf_f, sem, sem2)
    _drain_rows(entity_hbm, buf_c, sem, sem2)
    _drain_rows(entity_hbm, buf_d, sem, sem2)

    def cf_body(g, carry):
      e0 = g * _L
      acc = [zeros] * 5
      for l in range(_L):
        e = e0 + l
        u0 = buf_b[e, pl.ds(0, _L)]
        u1 = buf_b[e, pl.ds(_L, _L)]
        ipcf0 = buf_e[e, pl.ds(0, _L)] * buf_c[e, pl.ds(0, _L)]
        ipcf1 = buf_e[e, pl.ds(_L, _L)] * buf_c[e, pl.ds(_L, _L)]
        incf0 = buf_f[e, pl.ds(0, _L)] * buf_d[e, pl.ds(0, _L)]
        incf1 = buf_f[e, pl.ds(_L, _L)] * buf_d[e, pl.ds(_L, _L)]
        lane = iota == l
        vals = (jnp.sum(u0 * ipcf0 + u1 * ipcf1),
                jnp.sum(u0 * incf0 + u1 * incf1),
                jnp.sum(u0 * u0 + u1 * u1),
                jnp.sum(ipcf0 * ipcf0 + ipcf1 * ipcf1),
                jnp.sum(incf0 * incf0 + incf1 * incf1))
        acc = [jnp.where(lane, v, a) for v, a in zip(vals, acc)]
      for q in range(5):
        stage[8 + q, pl.ds(ce0 + e0, _L)] = acc[q]
      return carry

    lax.fori_loop(0, _CHUNK // _L, cf_body, jnp.int32(0))

  pltpu.sync_copy(stage, out_hbm.at[wid])


def _tc_stage2(x_ref, o_ref):
  def q(i):
    return x_ref[:, i, :]

  h2, r2, p2, n2 = q(0), q(1), q(2), q(3)
  hp, rp, hn, rn = q(4), q(5), q(6), q(7)
  ps, ns, u2, ip2, in2 = q(8), q(9), q(10), q(11), q(12)

  diff = 2.0 * (hp * lax.rsqrt(h2 * p2) + rp * lax.rsqrt(r2 * p2)
                - hn * lax.rsqrt(h2 * n2) - rn * lax.rsqrt(r2 * n2))
  kg_loss = jnp.mean(-jax.nn.log_sigmoid(diff))
  kg_total = kg_loss + _KG_LAMBDA * 2.0

  cf_loss = jnp.mean(-jnp.log(1e-10 + jax.nn.sigmoid(ps - ns)))
  cf_l2 = 0.5 * (jnp.mean(u2) + jnp.mean(ip2) + jnp.mean(in2))
  cf_total = cf_loss + _CF_LAMBDA * cf_l2

  o_ref[0, 0] = kg_total + cf_total


@jax.jit
def kernel(user_embed, item_embed, entity_embed, relation_embed,
           user_ids, item_pos_ids, item_neg_ids, h, r, pos_t, neg_t):
  ids = [jnp.asarray(a, jnp.int32)
         for a in (user_ids, item_pos_ids, item_neg_ids, h, r, pos_t, neg_t)]

  mesh = plsc.VectorSubcoreMesh(core_axis_name="c", subcore_axis_name="s")
  stage1 = pl.kernel(
      _sc_stage1,
      out_type=jax.ShapeDtypeStruct((_NW, _NQ, _BPW), jnp.float32),
      mesh=mesh,
      scratch_types=(
          [pltpu.VMEM((_BPW,), jnp.int32)] * 7
          + [pltpu.VMEM((_CHUNK, _D), jnp.float32)] * 6
          + [pltpu.VMEM((_NREL, _D), jnp.float32),
             pltpu.VMEM((_NQ, _BPW), jnp.float32),
             pltpu.SemaphoreType.DMA, pltpu.SemaphoreType.DMA]),
      compiler_params=pltpu.CompilerParams(needs_layout_passes=False),
  )
  inter = stage1(user_embed, item_embed, entity_embed, relation_embed, *ids)

  out = pl.pallas_call(
      _tc_stage2,
      out_shape=jax.ShapeDtypeStruct((1, 1), jnp.float32),
      out_specs=pl.BlockSpec(memory_space=pltpu.SMEM),
  )(inter)
  return out[0, 0]
